# baseline (speedup 1.0000x reference)
import jax
import jax.numpy as jnp
from jax.experimental import pallas as pl
from jax.experimental.pallas import tpu as pltpu

_B, _TM1, _V = 8, 2048, 4096
_TB = 256
_NB = (_B * _TM1) // _TB
_RPB = _TM1 // _TB


def _nll_kernel(length_ref, x_ref, t_ref, out_ref):
    i = pl.program_id(0)
    x = x_ref[0]
    m = jnp.max(x, axis=-1, keepdims=True)
    s = jnp.sum(jnp.exp(x - m), axis=-1, keepdims=True)
    tgt = t_ref[0]
    iota = jax.lax.broadcasted_iota(jnp.int32, (_TB, _V), 1)
    picked = jnp.sum(jnp.where(iota == tgt, x, 0.0), axis=-1, keepdims=True)
    nll = jnp.log(s) + m - picked
    w = jnp.where(length_ref[i // _RPB] > 0, 1.0, 0.0)
    out_ref[0, 0, 0] = w * jnp.sum(nll)
    out_ref[0, 1, 0] = w * _TB


def _reduce_kernel(p_ref, out_ref):
    p = p_ref[...]
    total = jnp.sum(p[:, 0, 0])
    count = jnp.sum(p[:, 1, 0])
    out_ref[0, 0] = total / jnp.maximum(count, 1.0)


def kernel(input, target, length):
    x = input.reshape(_NB, _TB, _V)
    tgt = target[:, 1:].reshape(_NB, _TB, 1)
    grid_spec = pltpu.PrefetchScalarGridSpec(
        num_scalar_prefetch=1,
        grid=(_NB,),
        in_specs=[
            pl.BlockSpec((1, _TB, _V), lambda i, *_: (i, 0, 0)),
            pl.BlockSpec((1, _TB, 1), lambda i, *_: (i, 0, 0)),
        ],
        out_specs=pl.BlockSpec((1, 2, 1), lambda i, *_: (i, 0, 0),
                               memory_space=pltpu.SMEM),
    )
    partials = pl.pallas_call(
        _nll_kernel,
        grid_spec=grid_spec,
        out_shape=jax.ShapeDtypeStruct((_NB, 2, 1), jnp.float32),
        compiler_params=pltpu.CompilerParams(
            dimension_semantics=("parallel",)),
    )(length, x, tgt)
    out = pl.pallas_call(
        _reduce_kernel,
        out_shape=jax.ShapeDtypeStruct((1, 1), jnp.float32),
        out_specs=pl.BlockSpec(memory_space=pltpu.SMEM),
    )(partials)
    return out[0, 0]


# speedup vs baseline: 1.1745x; 1.1745x over previous
import jax
import jax.numpy as jnp
from jax.experimental import pallas as pl
from jax.experimental.pallas import tpu as pltpu

_B, _TM1, _V = 8, 2048, 4096
_TB = 512
_NB = (_B * _TM1) // _TB
_RPB = _TM1 // _TB


def _nll_kernel(length_ref, x_ref, t_ref, out_ref, acc_ref):
    i = pl.program_id(0)

    @pl.when(i == 0)
    def _():
        acc_ref[0] = 0.0
        acc_ref[1] = 0.0

    x = x_ref[0]
    m = jnp.max(x, axis=-1, keepdims=True)
    s = jnp.sum(jnp.exp(x - m), axis=-1, keepdims=True)
    tgt = t_ref[0]
    iota = jax.lax.broadcasted_iota(jnp.int32, (_TB, _V), 1)
    picked = jnp.sum(jnp.where(iota == tgt, x, 0.0), axis=-1, keepdims=True)
    nll = jnp.log(s) + m - picked
    w = jnp.where(length_ref[i // _RPB] > 0, 1.0, 0.0)
    acc_ref[0] += w * jnp.sum(nll)
    acc_ref[1] += w * _TB

    @pl.when(i == _NB - 1)
    def _():
        out_ref[0, 0] = acc_ref[0] / jnp.maximum(acc_ref[1], 1.0)


def kernel(input, target, length):
    x = input.reshape(_NB, _TB, _V)
    tgt = target[:, 1:].reshape(_NB, _TB, 1)
    grid_spec = pltpu.PrefetchScalarGridSpec(
        num_scalar_prefetch=1,
        grid=(_NB,),
        in_specs=[
            pl.BlockSpec((1, _TB, _V), lambda i, *_: (i, 0, 0)),
            pl.BlockSpec((1, _TB, 1), lambda i, *_: (i, 0, 0)),
        ],
        out_specs=pl.BlockSpec((1, 1), lambda i, *_: (0, 0),
                               memory_space=pltpu.SMEM),
        scratch_shapes=[pltpu.SMEM((2,), jnp.float32)],
    )
    out = pl.pallas_call(
        _nll_kernel,
        grid_spec=grid_spec,
        out_shape=jax.ShapeDtypeStruct((1, 1), jnp.float32),
    )(length, x, tgt)
    return out[0, 0]


# speedup vs baseline: 1.2541x; 1.0678x over previous
import jax
import jax.numpy as jnp
from jax.experimental import pallas as pl
from jax.experimental.pallas import tpu as pltpu

_B, _TM1, _V = 8, 2048, 4096
_TB = 1024
_NB = (_B * _TM1) // _TB
_RPB = _TM1 // _TB


def _nll_kernel(length_ref, x_ref, t_ref, out_ref, acc_ref):
    i = pl.program_id(0)

    @pl.when(i == 0)
    def _():
        acc_ref[0] = 0.0
        acc_ref[1] = 0.0

    x = x_ref[0]
    m = jnp.max(x, axis=-1, keepdims=True)
    s = jnp.sum(jnp.exp(x - m), axis=-1, keepdims=True)
    tgt = t_ref[0]
    iota = jax.lax.broadcasted_iota(jnp.int32, (_TB, _V), 1)
    picked = jnp.sum(jnp.where(iota == tgt, x, 0.0), axis=-1, keepdims=True)
    nll = jnp.log(s) + m - picked
    w = jnp.where(length_ref[i // _RPB] > 0, 1.0, 0.0)
    acc_ref[0] += w * jnp.sum(nll)
    acc_ref[1] += w * _TB

    @pl.when(i == _NB - 1)
    def _():
        out_ref[0, 0] = acc_ref[0] / jnp.maximum(acc_ref[1], 1.0)


def kernel(input, target, length):
    x = input.reshape(_NB, _TB, _V)
    tgt = target[:, 1:].reshape(_NB, _TB, 1)
    grid_spec = pltpu.PrefetchScalarGridSpec(
        num_scalar_prefetch=1,
        grid=(_NB,),
        in_specs=[
            pl.BlockSpec((1, _TB, _V), lambda i, *_: (i, 0, 0)),
            pl.BlockSpec((1, _TB, 1), lambda i, *_: (i, 0, 0)),
        ],
        out_specs=pl.BlockSpec((1, 1), lambda i, *_: (0, 0),
                               memory_space=pltpu.SMEM),
        scratch_shapes=[pltpu.SMEM((2,), jnp.float32)],
    )
    out = pl.pallas_call(
        _nll_kernel,
        grid_spec=grid_spec,
        out_shape=jax.ShapeDtypeStruct((1, 1), jnp.float32),
    )(length, x, tgt)
    return out[0, 0]


# speedup vs baseline: 1.2659x; 1.0094x over previous
import jax
import jax.numpy as jnp
from jax.experimental import pallas as pl
from jax.experimental.pallas import tpu as pltpu

_B, _TM1, _V = 8, 2048, 4096
_TB = 1024
_H = _TB // 2
_NB = (_B * _TM1) // _TB
_RPB = _TM1 // _TB


def _half_nll_sum(x, tgt):
    m = jnp.max(x, axis=-1, keepdims=True)
    s = jnp.sum(jnp.exp(x - m), axis=-1, keepdims=True)
    iota = jax.lax.broadcasted_iota(jnp.int32, (_H, _V), 1)
    picked = jnp.sum(jnp.where(iota == tgt, x, 0.0), axis=-1, keepdims=True)
    return jnp.sum(jnp.log(s) + m - picked)


def _nll_kernel(length_ref, xa_ref, xb_ref, ta_ref, tb_ref, out_ref, acc_ref):
    i = pl.program_id(0)

    @pl.when(i == 0)
    def _():
        acc_ref[0] = 0.0
        acc_ref[1] = 0.0

    nll = _half_nll_sum(xa_ref[0], ta_ref[0]) + _half_nll_sum(xb_ref[0], tb_ref[0])
    w = jnp.where(length_ref[i // _RPB] > 0, 1.0, 0.0)
    acc_ref[0] += w * nll
    acc_ref[1] += w * _TB

    @pl.when(i == _NB - 1)
    def _():
        out_ref[0, 0] = acc_ref[0] / jnp.maximum(acc_ref[1], 1.0)


def kernel(input, target, length):
    x = input.reshape(_NB, _TB, _V)
    tgt = target[:, 1:].reshape(_NB, _TB, 1)
    grid_spec = pltpu.PrefetchScalarGridSpec(
        num_scalar_prefetch=1,
        grid=(_NB,),
        in_specs=[
            pl.BlockSpec((1, _H, _V), lambda i, *_: (i, 0, 0)),
            pl.BlockSpec((1, _H, _V), lambda i, *_: (i, 1, 0)),
            pl.BlockSpec((1, _H, 1), lambda i, *_: (i, 0, 0)),
            pl.BlockSpec((1, _H, 1), lambda i, *_: (i, 1, 0)),
        ],
        out_specs=pl.BlockSpec((1, 1), lambda i, *_: (0, 0),
                               memory_space=pltpu.SMEM),
        scratch_shapes=[pltpu.SMEM((2,), jnp.float32)],
    )
    out = pl.pallas_call(
        _nll_kernel,
        grid_spec=grid_spec,
        out_shape=jax.ShapeDtypeStruct((1, 1), jnp.float32),
    )(length, x, x, tgt, tgt)
    return out[0, 0]
